# baseline (speedup 1.0000x reference)
import functools

import jax
import jax.numpy as jnp
from jax import lax
from jax.experimental import pallas as pl
from jax.experimental.pallas import tpu as pltpu
from jax.experimental.pallas import tpu_sc as plsc

N_NODES = 10000
D = 128
N_PAD = 10240
NC = 2
NS = 16
NW = NC * NS
NB = 80
B = 128
NBP = 20
NCHW = 8
N0 = 4
N1 = NCHW - N0
NCH = NS * NCHW
EPT = NB * B
E_PAD = NW * EPT

_mesh = plsc.VectorSubcoreMesh(core_axis_name="c", subcore_axis_name="s")

_SEG = N_PAD // NS


@functools.partial(
    pl.kernel,
    out_type=jax.ShapeDtypeStruct((NC, N_PAD), jnp.float32),
    mesh=_mesh,
    scratch_types=[
        pltpu.VMEM((NB, B), jnp.int32),
        pltpu.VMEM((B,), jnp.float32),
        pltpu.VMEM((_SEG,), jnp.float32),
        pltpu.VMEM_SHARED((N_PAD,), jnp.float32),
    ],
)
def _deg_kernel(dst_hbm, ones_hbm, zeros_hbm, degpart_hbm, dstv, onesv, bounce,
                deg_sh):
    c = lax.axis_index("c")
    s = lax.axis_index("s")
    tid = c * NS + s
    lo = s * _SEG
    pltpu.sync_copy(zeros_hbm.at[pl.ds(lo, _SEG)], deg_sh.at[pl.ds(lo, _SEG)])
    pltpu.sync_copy(ones_hbm, onesv)
    pltpu.sync_copy(dst_hbm.at[tid], dstv)
    plsc.subcore_barrier()

    def body(j, carry):
        pltpu.sync_copy(onesv, deg_sh.at[dstv.at[j]], add=True)
        return carry

    lax.fori_loop(0, NB, body, 0)
    plsc.subcore_barrier()
    pltpu.sync_copy(deg_sh.at[pl.ds(lo, _SEG)], bounce)
    pltpu.sync_copy(bounce, degpart_hbm.at[c, pl.ds(lo, _SEG)])


@functools.partial(
    pl.kernel,
    out_type=jax.ShapeDtypeStruct((NC, N_PAD, D), jnp.float32),
    mesh=_mesh,
    scratch_types=[
        pltpu.VMEM((NBP, B), jnp.int32),
        pltpu.VMEM((NBP, B), jnp.int32),
        pltpu.VMEM((2, B, D), jnp.float32),
        pltpu.VMEM_SHARED((N_PAD, D), jnp.float32),
        pltpu.SemaphoreType.DMA,
        pltpu.SemaphoreType.DMA,
    ],
)
def _agg_kernel(hp_hbm, src_hbm, dst_hbm, zeros2_hbm, acc_hbm, srcv, dstv,
                rowbuf, acc_sh, g0, g1):
    c = lax.axis_index("c")
    s = lax.axis_index("s")
    tid = c * NS + s
    lo = s * _SEG
    gsem = (g0, g1)
    pltpu.sync_copy(zeros2_hbm.at[pl.ds(lo, _SEG)], acc_sh.at[pl.ds(lo, _SEG)])
    plsc.subcore_barrier()

    def gather(j, p):
        pltpu.async_copy(hp_hbm.at[srcv.at[j]], rowbuf.at[p], gsem[p])

    def gather_wait(j, p):
        pltpu.make_async_copy(hp_hbm.at[srcv.at[j]], rowbuf.at[p],
                              gsem[p]).wait()

    def run_chunk(ck, carry):
        pltpu.sync_copy(src_hbm.at[ck], srcv)
        pltpu.sync_copy(dst_hbm.at[ck], dstv)
        gather(0, 0)

        def body(g, c2):
            for u in range(2):
                j = g * 2 + u
                p = u
                q = 1 - u

                @pl.when(j + 1 < NBP)
                def _():
                    gather(j + 1, q)

                gather_wait(j, p)
                pltpu.sync_copy(rowbuf.at[p], acc_sh.at[dstv.at[j]],
                                add=True)

            return c2

        lax.fori_loop(0, NBP // 2, body, 0)
        return carry

    @pl.when(c == 0)
    def _():
        lax.fori_loop(s * N0, s * N0 + N0, run_chunk, 0)

    @pl.when(c == 1)
    def _():
        base = NS * N0 + s * N1
        lax.fori_loop(base, base + N1, run_chunk, 0)

    plsc.subcore_barrier()

    pltpu.sync_copy(acc_sh.at[pl.ds(lo, _SEG)],
                    acc_hbm.at[c, pl.ds(lo, _SEG)])


def _tc1_body(degp, x_r, w_r, out_r):
    dis = lax.rsqrt(degp[0, :] + degp[1, :] + 1.0)
    h = jnp.dot(x_r[...], w_r[...], preferred_element_type=jnp.float32)
    out_r[...] = h * dis[:, None]


def _tc2_body(degp, a0, a1, hp1, b1r, w2r, out_r):
    dis = lax.rsqrt(degp[0, :] + degp[1, :] + 1.0)
    pre = (a0[...] + a1[...] + hp1[...]) * dis[:, None] + b1r[...]
    h2 = jnp.maximum(pre, 0.0)
    out_r[...] = jnp.dot(h2, w2r[...],
                         preferred_element_type=jnp.float32) * dis[:, None]


def _tc3_body(degp, a0, a1, hp2, b2r, out_r):
    dis = lax.rsqrt(degp[0, :] + degp[1, :] + 1.0)
    out_r[...] = (a0[...] + a1[...] + hp2[...]) * dis[:, None] + b2r[...]


_RB = 1024
_NRB = N_PAD // _RB
_GRID = (_NRB,)
_degp_spec = pl.BlockSpec((2, _RB), lambda i: (0, i))
_row_spec = pl.BlockSpec((_RB, D), lambda i: (i, 0))
_w_spec = pl.BlockSpec((D, D), lambda i: (0, 0))
_b_spec = pl.BlockSpec((1, D), lambda i: (0, 0))
_out_sds = jax.ShapeDtypeStruct((N_PAD, D), jnp.float32)

_tc1 = pl.pallas_call(
    _tc1_body, grid=_GRID,
    in_specs=[_degp_spec, _row_spec, _w_spec],
    out_specs=_row_spec, out_shape=_out_sds)

_tc2 = pl.pallas_call(
    _tc2_body, grid=_GRID,
    in_specs=[_degp_spec, _row_spec, _row_spec, _row_spec, _b_spec, _w_spec],
    out_specs=_row_spec, out_shape=_out_sds)

_tc3 = pl.pallas_call(
    _tc3_body, grid=_GRID,
    in_specs=[_degp_spec, _row_spec, _row_spec, _row_spec, _b_spec],
    out_specs=_row_spec, out_shape=_out_sds)


def kernel(x, edge_index, W1, b1, W2, b2):
    ei = edge_index.astype(jnp.int32)
    n_edges = ei.shape[1]
    pad = E_PAD - n_edges
    sink_src = jnp.full((pad,), N_NODES, dtype=jnp.int32)
    sink_dst = jnp.arange(pad, dtype=jnp.int32) % N_NODES
    src_p = jnp.concatenate([ei[0], sink_src]).reshape(NCH, NBP, B)
    dst_p = jnp.concatenate([ei[1], sink_dst]).reshape(NCH, NBP, B)
    dst_deg = jnp.concatenate([ei[1], sink_src]).reshape(NW, NB, B)
    x_pad = jnp.concatenate(
        [x, jnp.zeros((N_PAD - N_NODES, D), jnp.float32)], axis=0)
    ones_b = jnp.ones((B,), jnp.float32)
    zeros1 = jnp.zeros((N_PAD,), jnp.float32)
    zeros2 = jnp.zeros((N_PAD, D), jnp.float32)
    b1r = b1.reshape(1, D).astype(jnp.float32)
    b2r = b2.reshape(1, D).astype(jnp.float32)

    degpart = _deg_kernel(dst_deg, ones_b, zeros1)
    hp1 = _tc1(degpart, x_pad, W1)
    acc1 = _agg_kernel(hp1, src_p, dst_p, zeros2)
    hp2 = _tc2(degpart, acc1[0], acc1[1], hp1, b1r, W2)
    acc2 = _agg_kernel(hp2, src_p, dst_p, zeros2)
    out = _tc3(degpart, acc2[0], acc2[1], hp2, b2r)
    return out[:N_NODES]


# speedup vs baseline: 1.0446x; 1.0446x over previous
import functools

import jax
import jax.numpy as jnp
from jax import lax
from jax.experimental import pallas as pl
from jax.experimental.pallas import tpu as pltpu
from jax.experimental.pallas import tpu_sc as plsc

N_NODES = 10000
D = 128
N_PAD = 10240
NC = 2
NS = 16
NW = NC * NS
NB = 80
B = 128
NBP = 40
NCHW = 4
N0 = 2
N1 = NCHW - N0
NCH = NS * NCHW
EPT = NB * B
E_PAD = NW * EPT

_mesh = plsc.VectorSubcoreMesh(core_axis_name="c", subcore_axis_name="s")

_SEG = N_PAD // NS


@functools.partial(
    pl.kernel,
    out_type=jax.ShapeDtypeStruct((NC, N_PAD), jnp.float32),
    mesh=_mesh,
    scratch_types=[
        pltpu.VMEM((NB, B), jnp.int32),
        pltpu.VMEM((B,), jnp.float32),
        pltpu.VMEM((_SEG,), jnp.float32),
        pltpu.VMEM_SHARED((N_PAD,), jnp.float32),
    ],
)
def _deg_kernel(dst_hbm, ones_hbm, zeros_hbm, degpart_hbm, dstv, onesv, bounce,
                deg_sh):
    c = lax.axis_index("c")
    s = lax.axis_index("s")
    tid = c * NS + s
    lo = s * _SEG
    pltpu.sync_copy(zeros_hbm.at[pl.ds(lo, _SEG)], deg_sh.at[pl.ds(lo, _SEG)])
    pltpu.sync_copy(ones_hbm, onesv)
    pltpu.sync_copy(dst_hbm.at[tid], dstv)
    plsc.subcore_barrier()

    def body(j, carry):
        pltpu.sync_copy(onesv, deg_sh.at[dstv.at[j]], add=True)
        return carry

    lax.fori_loop(0, NB, body, 0)
    plsc.subcore_barrier()
    pltpu.sync_copy(deg_sh.at[pl.ds(lo, _SEG)], bounce)
    pltpu.sync_copy(bounce, degpart_hbm.at[c, pl.ds(lo, _SEG)])


@functools.partial(
    pl.kernel,
    out_type=jax.ShapeDtypeStruct((NC, N_PAD, D), jnp.float32),
    mesh=_mesh,
    scratch_types=[
        pltpu.VMEM((NBP, B), jnp.int32),
        pltpu.VMEM((NBP, B), jnp.int32),
        pltpu.VMEM((2, B, D), jnp.float32),
        pltpu.VMEM_SHARED((N_PAD, D), jnp.float32),
        pltpu.SemaphoreType.DMA,
        pltpu.SemaphoreType.DMA,
    ],
)
def _agg_kernel(hp_hbm, src_hbm, dst_hbm, zeros2_hbm, acc_hbm, srcv, dstv,
                rowbuf, acc_sh, g0, g1):
    c = lax.axis_index("c")
    s = lax.axis_index("s")
    tid = c * NS + s
    lo = s * _SEG
    gsem = (g0, g1)
    pltpu.sync_copy(zeros2_hbm.at[pl.ds(lo, _SEG)], acc_sh.at[pl.ds(lo, _SEG)])
    plsc.subcore_barrier()

    def gather(j, p):
        pltpu.async_copy(hp_hbm.at[srcv.at[j]], rowbuf.at[p], gsem[p])

    def gather_wait(j, p):
        pltpu.make_async_copy(hp_hbm.at[srcv.at[j]], rowbuf.at[p],
                              gsem[p]).wait()

    def run_chunk(ck, carry):
        pltpu.sync_copy(src_hbm.at[ck], srcv)
        pltpu.sync_copy(dst_hbm.at[ck], dstv)
        gather(0, 0)

        def body(g, c2):
            for u in range(2):
                j = g * 2 + u
                p = u
                q = 1 - u

                @pl.when(j + 1 < NBP)
                def _():
                    gather(j + 1, q)

                gather_wait(j, p)
                pltpu.sync_copy(rowbuf.at[p], acc_sh.at[dstv.at[j]],
                                add=True)

            return c2

        lax.fori_loop(0, NBP // 2, body, 0)
        return carry

    @pl.when(c == 0)
    def _():
        lax.fori_loop(s * N0, s * N0 + N0, run_chunk, 0)

    @pl.when(c == 1)
    def _():
        base = NS * N0 + s * N1
        lax.fori_loop(base, base + N1, run_chunk, 0)

    plsc.subcore_barrier()

    pltpu.sync_copy(acc_sh.at[pl.ds(lo, _SEG)],
                    acc_hbm.at[c, pl.ds(lo, _SEG)])


def _tc1_body(degp, x_r, w_r, out_r):
    dis = lax.rsqrt(degp[0, :] + degp[1, :] + 1.0)
    h = jnp.dot(x_r[...], w_r[...], preferred_element_type=jnp.float32)
    out_r[...] = h * dis[:, None]


def _tc2_body(degp, a0, a1, hp1, b1r, w2r, out_r):
    dis = lax.rsqrt(degp[0, :] + degp[1, :] + 1.0)
    pre = (a0[...] + a1[...] + hp1[...]) * dis[:, None] + b1r[...]
    h2 = jnp.maximum(pre, 0.0)
    out_r[...] = jnp.dot(h2, w2r[...],
                         preferred_element_type=jnp.float32) * dis[:, None]


def _tc3_body(degp, a0, a1, hp2, b2r, out_r):
    dis = lax.rsqrt(degp[0, :] + degp[1, :] + 1.0)
    out_r[...] = (a0[...] + a1[...] + hp2[...]) * dis[:, None] + b2r[...]


_RB = 1024
_NRB = N_PAD // _RB
_GRID = (_NRB,)
_degp_spec = pl.BlockSpec((2, _RB), lambda i: (0, i))
_row_spec = pl.BlockSpec((_RB, D), lambda i: (i, 0))
_w_spec = pl.BlockSpec((D, D), lambda i: (0, 0))
_b_spec = pl.BlockSpec((1, D), lambda i: (0, 0))
_out_sds = jax.ShapeDtypeStruct((N_PAD, D), jnp.float32)

_tc1 = pl.pallas_call(
    _tc1_body, grid=_GRID,
    in_specs=[_degp_spec, _row_spec, _w_spec],
    out_specs=_row_spec, out_shape=_out_sds)

_tc2 = pl.pallas_call(
    _tc2_body, grid=_GRID,
    in_specs=[_degp_spec, _row_spec, _row_spec, _row_spec, _b_spec, _w_spec],
    out_specs=_row_spec, out_shape=_out_sds)

_tc3 = pl.pallas_call(
    _tc3_body, grid=_GRID,
    in_specs=[_degp_spec, _row_spec, _row_spec, _row_spec, _b_spec],
    out_specs=_row_spec, out_shape=_out_sds)


def kernel(x, edge_index, W1, b1, W2, b2):
    ei = edge_index.astype(jnp.int32)
    n_edges = ei.shape[1]
    pad = E_PAD - n_edges
    sink_src = jnp.full((pad,), N_NODES, dtype=jnp.int32)
    sink_dst = jnp.arange(pad, dtype=jnp.int32) % N_NODES
    src_p = jnp.concatenate([ei[0], sink_src]).reshape(NCH, NBP, B)
    dst_p = jnp.concatenate([ei[1], sink_dst]).reshape(NCH, NBP, B)
    dst_deg = jnp.concatenate([ei[1], sink_src]).reshape(NW, NB, B)
    x_pad = jnp.concatenate(
        [x, jnp.zeros((N_PAD - N_NODES, D), jnp.float32)], axis=0)
    ones_b = jnp.ones((B,), jnp.float32)
    zeros1 = jnp.zeros((N_PAD,), jnp.float32)
    zeros2 = jnp.zeros((N_PAD, D), jnp.float32)
    b1r = b1.reshape(1, D).astype(jnp.float32)
    b2r = b2.reshape(1, D).astype(jnp.float32)

    degpart = _deg_kernel(dst_deg, ones_b, zeros1)
    hp1 = _tc1(degpart, x_pad, W1)
    acc1 = _agg_kernel(hp1, src_p, dst_p, zeros2)
    hp2 = _tc2(degpart, acc1[0], acc1[1], hp1, b1r, W2)
    acc2 = _agg_kernel(hp2, src_p, dst_p, zeros2)
    out = _tc3(degpart, acc2[0], acc2[1], hp2, b2r)
    return out[:N_NODES]
